# Initial kernel scaffold; baseline (speedup 1.0000x reference)
import jax
import jax.numpy as jnp
from jax.experimental import pallas as pl


def kernel(x, edge_index, edge_weights, batch, W, b):
    raise NotImplementedError("write your pallas kernel here")


# speedup vs baseline: 63.6554x; 63.6554x over previous
import functools

import jax
import jax.numpy as jnp
from jax import lax
from jax.experimental import pallas as pl
from jax.experimental.pallas import tpu as pltpu
from jax.experimental.pallas import tpu_sc as plsc

_N = 10000
_E = 320000
_D = 128
_C = 16
_G = 16

_NC = 2
_NS = 16
_NW = _NC * _NS
_L = 16

_EA_PER = _E // _NW
_EP = 331776
_EC_PER = _EP // _NW
_ROWS = _EC_PER // 128
_COEFF = _N * _G
_CO_PER = _COEFF // _NS

_mesh = plsc.VectorSubcoreMesh(
    core_axis_name="c", subcore_axis_name="s", num_cores=_NC, num_subcores=_NS
)
_sc_params = pltpu.CompilerParams(
    needs_layout_passes=False, use_tc_tiling_on_sc=False
)


def _deg_body(dst_hbm, w_hbm, part_hbm, dstv, wv, degv):
    c = lax.axis_index("c")
    s = lax.axis_index("s")
    wid = c * _NS + s
    base = wid * _EA_PER
    pltpu.sync_copy(dst_hbm.at[pl.ds(base, _EA_PER)], dstv)
    pltpu.sync_copy(w_hbm.at[pl.ds(base, _EA_PER)], wv)

    zeros = jnp.zeros((_L,), jnp.float32)

    def zb(i, carry):
        degv[pl.ds(i * _L, _L)] = zeros
        return carry

    lax.fori_loop(0, _N // _L, zb, 0)

    def eb(i, carry):
        d = dstv[pl.ds(i * _L, _L)]
        w16 = wv[pl.ds(i * _L, _L)]
        plsc.addupdate_scatter(degv, [d], w16)
        return carry

    lax.fori_loop(0, _EA_PER // _L, eb, 0)
    pltpu.sync_copy(degv, part_hbm.at[wid])


_deg_kernel = pl.kernel(
    _deg_body,
    out_type=jax.ShapeDtypeStruct((_NW, _N), jnp.float32),
    mesh=_mesh,
    compiler_params=_sc_params,
    scratch_types=[
        pltpu.VMEM((_EA_PER,), jnp.int32),
        pltpu.VMEM((_EA_PER,), jnp.float32),
        pltpu.VMEM((_N,), jnp.float32),
    ],
)


def _prep_body(part_ref, batch_ref, dinv_ref, invc_ref, bmask_ref):
    deg = jnp.sum(part_ref[...], axis=0, keepdims=True) + 1.0
    dinv_ref[...] = lax.rsqrt(deg)
    bv = jnp.broadcast_to(batch_ref[...], (_G, _N))
    gi = lax.broadcasted_iota(jnp.int32, (_G, _N), 0)
    cnt = jnp.sum((bv == gi).astype(jnp.float32), axis=1, keepdims=True)
    pos = cnt > 0.0
    invc_ref[...] = jnp.where(pos, 1.0 / jnp.maximum(cnt, 1.0), 0.0)
    bmask_ref[...] = jnp.where(pos, 1.0, 0.0)


_prep_kernel = pl.pallas_call(
    _prep_body,
    out_shape=(
        jax.ShapeDtypeStruct((1, _N), jnp.float32),
        jax.ShapeDtypeStruct((_G, 1), jnp.float32),
        jax.ShapeDtypeStruct((_G, 1), jnp.float32),
    ),
)


def _coeff_body(src_hbm, dst_hbm, w_hbm, dinv_hbm, batch_hbm, out_hbm,
                srcv, dstv, wv, dinv_v, batch_v, idx_st, val_st, zv, coeff_sp):
    c = lax.axis_index("c")
    s = lax.axis_index("s")
    wid = c * _NS + s
    base = wid * _EC_PER
    pltpu.sync_copy(src_hbm.at[pl.ds(base, _EC_PER)], srcv)
    pltpu.sync_copy(dst_hbm.at[pl.ds(base, _EC_PER)], dstv)
    pltpu.sync_copy(w_hbm.at[pl.ds(base, _EC_PER)], wv)
    pltpu.sync_copy(dinv_hbm, dinv_v)
    pltpu.sync_copy(batch_hbm, batch_v)

    zeros = jnp.zeros((_L,), jnp.float32)

    def zb(i, carry):
        zv[pl.ds(i * _L, _L)] = zeros
        return carry

    lax.fori_loop(0, _CO_PER // _L, zb, 0)
    pltpu.sync_copy(zv, coeff_sp.at[pl.ds(s * _CO_PER, _CO_PER)])
    plsc.subcore_barrier()

    def rb(r, carry):
        for u in range(8):
            off = r * 128 + u * _L
            sv = srcv[pl.ds(off, _L)]
            dv = dstv[pl.ds(off, _L)]
            w16 = wv[pl.ds(off, _L)]
            dis = plsc.load_gather(dinv_v, [sv])
            did = plsc.load_gather(dinv_v, [dv])
            g = plsc.load_gather(batch_v, [dv])
            idx_st[r, pl.ds(u * _L, _L)] = sv * _G + g
            val_st[r, pl.ds(u * _L, _L)] = dis * w16 * did
        pltpu.sync_copy(val_st.at[r], coeff_sp.at[idx_st.at[r]], add=True)
        return carry

    lax.fori_loop(0, _ROWS, rb, 0)
    plsc.subcore_barrier()
    pltpu.sync_copy(
        coeff_sp.at[pl.ds(s * _CO_PER, _CO_PER)],
        out_hbm.at[c, pl.ds(s * _CO_PER, _CO_PER)],
    )


_coeff_kernel = pl.kernel(
    _coeff_body,
    out_type=jax.ShapeDtypeStruct((_NC, _COEFF), jnp.float32),
    mesh=_mesh,
    compiler_params=_sc_params,
    scratch_types=[
        pltpu.VMEM((_EC_PER,), jnp.int32),
        pltpu.VMEM((_EC_PER,), jnp.int32),
        pltpu.VMEM((_EC_PER,), jnp.float32),
        pltpu.VMEM((_N,), jnp.float32),
        pltpu.VMEM((_N,), jnp.int32),
        pltpu.VMEM((_ROWS, 128), jnp.int32),
        pltpu.VMEM((_ROWS, 128), jnp.float32),
        pltpu.VMEM((_CO_PER,), jnp.float32),
        pltpu.VMEM_SHARED((_COEFF,), jnp.float32),
    ],
)


def _pool_body(c0_ref, c1_ref, x_ref, w_ref, bb_ref, invc_ref, bmask_ref,
               out_ref):
    c2 = c0_ref[...] + c1_ref[...]
    s = lax.dot_general(c2, x_ref[...], (((0,), (0,)), ((), ())),
                        preferred_element_type=jnp.float32)
    p = lax.dot_general(s, w_ref[...], (((1,), (0,)), ((), ())),
                        preferred_element_type=jnp.float32)
    out_ref[...] = p * invc_ref[...] + bb_ref[...] * bmask_ref[...]


_pool_kernel = pl.pallas_call(
    _pool_body,
    out_shape=jax.ShapeDtypeStruct((_G, _C), jnp.float32),
)


def kernel(x, edge_index, edge_weights, batch, W, b):
    src = edge_index[0].astype(jnp.int32)
    dst = edge_index[1].astype(jnp.int32)
    batch32 = batch.astype(jnp.int32)
    w = edge_weights.astype(jnp.float32)

    part = _deg_kernel(dst, w)
    dinv2d, invc, bmask = _prep_kernel(part, batch32.reshape(1, _N))
    dinv = dinv2d.reshape(_N)

    loop = jnp.arange(_N, dtype=jnp.int32)
    npad = _EP - (_E + _N)
    zpad_i = jnp.zeros((npad,), jnp.int32)
    srcp = jnp.concatenate([src, loop, zpad_i])
    dstp = jnp.concatenate([dst, loop, zpad_i])
    wp = jnp.concatenate(
        [w, jnp.ones((_N,), jnp.float32), jnp.zeros((npad,), jnp.float32)]
    )

    coeffp = _coeff_kernel(srcp, dstp, wp, dinv, batch32)
    cp = coeffp.reshape(_NC, _N, _G)

    return _pool_kernel(cp[0], cp[1], x, W, b.reshape(1, _C), invc, bmask)


# speedup vs baseline: 103.5167x; 1.6262x over previous
import jax
import jax.numpy as jnp
from jax import lax
from jax.experimental import pallas as pl
from jax.experimental.pallas import tpu as pltpu
from jax.experimental.pallas import tpu_sc as plsc

_N = 10000
_NP = 10240
_E = 320000
_D = 128
_C = 16
_G = 16

_NC = 2
_NS = 16
_NW = _NC * _NS
_L = 16

_E_PER = _E // _NW
_EV = _E_PER // _L
_NPT = _NP // _NS
_SLOTS = 648
_ROWS = _SLOTS // 8
_COEFF = _N * _G
_CO_PER = _COEFF // _NS
_PIPE = 4

_mesh = plsc.VectorSubcoreMesh(
    core_axis_name="c", subcore_axis_name="s", num_cores=_NC, num_subcores=_NS
)
_sc_params = pltpu.CompilerParams(
    needs_layout_passes=False, use_tc_tiling_on_sc=False
)

def _rsqrt16(x):
    magic = jnp.full((_L,), 0x5F3759DF, jnp.int32)
    y = plsc.bitcast(magic - (plsc.bitcast(x, jnp.int32) >> 1), jnp.float32)
    for _ in range(3):
        y = y * (1.5 - 0.5 * x * y * y)
    return y


def _deg_body(dst_hbm, w_hbm, part_hbm, dstv, wv, degv, sem):
    c = lax.axis_index("c")
    s = lax.axis_index("s")
    wid = c * _NS + s
    base = wid * _E_PER
    cp1 = pltpu.async_copy(dst_hbm.at[pl.ds(base, _E_PER)], dstv, sem)
    cp2 = pltpu.async_copy(w_hbm.at[pl.ds(base, _E_PER)], wv, sem)

    zeros = jnp.zeros((_L,), jnp.float32)

    def zb(i, carry):
        degv[pl.ds(i * _L, _L)] = zeros
        return carry

    lax.fori_loop(0, _NP // _L, zb, 0)
    cp1.wait()
    cp2.wait()

    def eb(i, carry):
        d = dstv[pl.ds(i * _L, _L)]
        w16 = wv[pl.ds(i * _L, _L)]
        plsc.addupdate_scatter(degv, [d], w16)
        return carry

    lax.fori_loop(0, _EV, eb, 0)
    pltpu.sync_copy(degv, part_hbm.at[wid])


_deg_kernel = pl.kernel(
    _deg_body,
    out_type=jax.ShapeDtypeStruct((_NW, _NP), jnp.float32),
    mesh=_mesh,
    compiler_params=_sc_params,
    scratch_types=[
        pltpu.VMEM((_E_PER,), jnp.int32),
        pltpu.VMEM((_E_PER,), jnp.float32),
        pltpu.VMEM((_NP,), jnp.float32),
        pltpu.SemaphoreType.DMA,
    ],
)


def _coeff_body(src_hbm, dst_hbm, w_hbm, part_hbm, batch_hbm, out_hbm,
                srcv, dstv, wv, pb, dinv_t, dinv_v, batch_v, idx_st, val_st,
                zv, dinv_sp, coeff_sp, sem_in, sem_sc):
    c = lax.axis_index("c")
    s = lax.axis_index("s")
    wid = c * _NS + s
    base = wid * _E_PER
    nbase = s * _NPT

    cp1 = pltpu.async_copy(src_hbm.at[pl.ds(base, _E_PER)], srcv, sem_in)
    cp2 = pltpu.async_copy(dst_hbm.at[pl.ds(base, _E_PER)], dstv, sem_in)
    cp3 = pltpu.async_copy(w_hbm.at[pl.ds(base, _E_PER)], wv, sem_in)
    cp4 = pltpu.async_copy(batch_hbm, batch_v, sem_in)
    cp5 = pltpu.async_copy(part_hbm.at[:, pl.ds(nbase, _NPT)], pb, sem_in)

    zeros = jnp.zeros((_L,), jnp.float32)

    def zb(i, carry):
        zv[pl.ds(i * _L, _L)] = zeros
        return carry

    lax.fori_loop(0, _CO_PER // _L, zb, 0)
    cp1.wait()
    cp2.wait()
    cp3.wait()
    cp4.wait()
    cp5.wait()

    def pk(k, carry):
        col = pl.ds(k * _L, _L)
        acc = pb[0, col]
        for p in range(1, _NW):
            acc = acc + pb[p, col]
        dinv_t[col] = _rsqrt16(acc + 1.0)
        return carry

    lax.fori_loop(0, _NPT // _L, pk, 0)
    pltpu.sync_copy(dinv_t, dinv_sp.at[pl.ds(nbase, _NPT)])
    pltpu.sync_copy(zv, coeff_sp.at[pl.ds(s * _CO_PER, _CO_PER)])
    plsc.subcore_barrier()
    pltpu.sync_copy(dinv_sp, dinv_v)

    def fire(r):
        return pltpu.async_copy(
            val_st.at[r], coeff_sp.at[idx_st.at[r]], sem_sc, add=True
        )

    def drain(r):
        pltpu.make_async_copy(
            val_st.at[r], coeff_sp.at[idx_st.at[r]], sem_sc
        ).wait()

    def edge_slot(r, u, off):
        sv = srcv[pl.ds(off, _L)]
        dv = dstv[pl.ds(off, _L)]
        w16 = wv[pl.ds(off, _L)]
        dis = plsc.load_gather(dinv_v, [sv])
        did = plsc.load_gather(dinv_v, [dv])
        g = plsc.load_gather(batch_v, [dv])
        idx_st[r, pl.ds(u * _L, _L)] = sv * _G + g
        val_st[r, pl.ds(u * _L, _L)] = dis * w16 * did

    def rb(r, carry):
        for u in range(8):
            edge_slot(r, u, r * 128 + u * _L)
        fire(r)

        @pl.when(r >= _PIPE)
        def _():
            drain(r - _PIPE)

        return carry

    lax.fori_loop(0, 78, rb, 0)

    edge_slot(78, 0, 624 * _L)
    iota16 = lax.iota(jnp.int32, _L)
    for j in range(20):
        q = _EV + j
        r, u = q // 8, q % 8
        v = wid + 32 * j
        n0 = jnp.minimum(v, _EV - 1) * _L
        g = batch_v[pl.ds(n0, _L)]
        y = dinv_v[pl.ds(n0, _L)]
        val = y * y
        if j == 19:
            val = val * jnp.where(v < _EV, 1.0, 0.0)
        idx_st[r, pl.ds(u * _L, _L)] = (n0 + iota16) * _G + g
        val_st[r, pl.ds(u * _L, _L)] = val
    for q in range(_EV + 20, _SLOTS):
        idx_st[q // 8, pl.ds((q % 8) * _L, _L)] = jnp.zeros((_L,), jnp.int32)
        val_st[q // 8, pl.ds((q % 8) * _L, _L)] = zeros
    for r in (78, 79, 80):
        fire(r)
    for r in range(78 - _PIPE, _ROWS):
        drain(r)

    plsc.subcore_barrier()
    pltpu.sync_copy(
        coeff_sp.at[pl.ds(s * _CO_PER, _CO_PER)],
        out_hbm.at[c, pl.ds(s * _CO_PER, _CO_PER)],
    )


_coeff_kernel = pl.kernel(
    _coeff_body,
    out_type=jax.ShapeDtypeStruct((_NC, _COEFF), jnp.float32),
    mesh=_mesh,
    compiler_params=_sc_params,
    scratch_types=[
        pltpu.VMEM((_E_PER,), jnp.int32),
        pltpu.VMEM((_E_PER,), jnp.int32),
        pltpu.VMEM((_E_PER,), jnp.float32),
        pltpu.VMEM((_NW, _NPT), jnp.float32),
        pltpu.VMEM((_NPT,), jnp.float32),
        pltpu.VMEM((_NP,), jnp.float32),
        pltpu.VMEM((_N,), jnp.int32),
        pltpu.VMEM((_ROWS, 128), jnp.int32),
        pltpu.VMEM((_ROWS, 128), jnp.float32),
        pltpu.VMEM((_CO_PER,), jnp.float32),
        pltpu.VMEM_SHARED((_NP,), jnp.float32),
        pltpu.VMEM_SHARED((_COEFF,), jnp.float32),
        pltpu.SemaphoreType.DMA,
        pltpu.SemaphoreType.DMA,
    ],
)


def _pool_body(c_ref, x_ref, w_ref, bb_ref, batch_ref, out_ref):
    bv = jnp.broadcast_to(batch_ref[...], (_G, _N))
    gi = lax.broadcasted_iota(jnp.int32, (_G, _N), 0)
    cnt = jnp.sum((bv == gi).astype(jnp.float32), axis=1, keepdims=True)
    pos = cnt > 0.0
    invc = jnp.where(pos, 1.0 / jnp.maximum(cnt, 1.0), 0.0)
    bmask = jnp.where(pos, 1.0, 0.0)
    c2 = c_ref[0] + c_ref[1]
    s = lax.dot_general(c2, x_ref[...], (((0,), (0,)), ((), ())),
                        preferred_element_type=jnp.float32)
    p = lax.dot_general(s, w_ref[...], (((1,), (0,)), ((), ())),
                        preferred_element_type=jnp.float32)
    out_ref[...] = p * invc + bb_ref[...] * bmask


_pool_kernel = pl.pallas_call(
    _pool_body,
    out_shape=jax.ShapeDtypeStruct((_G, _C), jnp.float32),
)


def kernel(x, edge_index, edge_weights, batch, W, b):
    src = edge_index[0].astype(jnp.int32)
    dst = edge_index[1].astype(jnp.int32)
    batch32 = batch.astype(jnp.int32)
    w = edge_weights.astype(jnp.float32)

    part = _deg_kernel(dst, w)
    coeffp = _coeff_kernel(src, dst, w, part, batch32)
    return _pool_kernel(coeffp.reshape(_NC, _N, _G), x, W,
                        b.reshape(1, _C), batch32.reshape(1, _N))


# speedup vs baseline: 135.6014x; 1.3099x over previous
import jax
import jax.numpy as jnp
from jax import lax
from jax.experimental import pallas as pl
from jax.experimental.pallas import tpu as pltpu
from jax.experimental.pallas import tpu_sc as plsc

_N = 10000
_NP = 10240
_E = 320000
_D = 128
_C = 16
_G = 16

_NC = 2
_NS = 16
_NW = _NC * _NS
_L = 16

_E_PER = _E // _NW
_EV = _E_PER // _L
_NPT = _NP // _NS
_SLOTS = 648
_ROWS = _SLOTS // 8
_COEFF = _N * _G
_CO_PER = _COEFF // _NS
_PIPE = 4

_mesh = plsc.VectorSubcoreMesh(
    core_axis_name="c", subcore_axis_name="s", num_cores=_NC, num_subcores=_NS
)
_sc_params = pltpu.CompilerParams(
    needs_layout_passes=False, use_tc_tiling_on_sc=False
)

def _rsqrt16(x):
    magic = jnp.full((_L,), 0x5F3759DF, jnp.int32)
    y = plsc.bitcast(magic - (plsc.bitcast(x, jnp.int32) >> 1), jnp.float32)
    for _ in range(3):
        y = y * (1.5 - 0.5 * x * y * y)
    return y


def _deg_body(ei_hbm, w_hbm, part_hbm, dstv, wv, degv, sem):
    c = lax.axis_index("c")
    s = lax.axis_index("s")
    wid = c * _NS + s
    base = wid * _E_PER
    cp1 = pltpu.async_copy(ei_hbm.at[1, pl.ds(base, _E_PER)], dstv, sem)
    cp2 = pltpu.async_copy(w_hbm.at[pl.ds(base, _E_PER)], wv, sem)

    zeros = jnp.zeros((_L,), jnp.float32)

    def zb(i, carry):
        degv[pl.ds(i * _L, _L)] = zeros
        return carry

    lax.fori_loop(0, _NP // _L, zb, 0)
    cp1.wait()
    cp2.wait()

    def eb(i, carry):
        d = dstv[pl.ds(i * _L, _L)]
        w16 = wv[pl.ds(i * _L, _L)]
        plsc.addupdate_scatter(degv, [d], w16)
        return carry

    lax.fori_loop(0, _EV, eb, 0)
    pltpu.sync_copy(degv, part_hbm.at[wid])


_deg_kernel = pl.kernel(
    _deg_body,
    out_type=jax.ShapeDtypeStruct((_NW, _NP), jnp.float32),
    mesh=_mesh,
    compiler_params=_sc_params,
    scratch_types=[
        pltpu.VMEM((_E_PER,), jnp.int32),
        pltpu.VMEM((_E_PER,), jnp.float32),
        pltpu.VMEM((_NP,), jnp.float32),
        pltpu.SemaphoreType.DMA,
    ],
)


def _coeff_body(ei_hbm, w_hbm, part_hbm, batch_hbm, out_hbm,
                srcv, dstv, wv, pb, dinv_t, dinv_v, batch_v, idx_st, val_st,
                zv, dinv_sp, coeff_sp, sem_in, sem_sc):
    c = lax.axis_index("c")
    s = lax.axis_index("s")
    wid = c * _NS + s
    base = wid * _E_PER
    nbase = s * _NPT

    cp1 = pltpu.async_copy(ei_hbm.at[0, pl.ds(base, _E_PER)], srcv, sem_in)
    cp2 = pltpu.async_copy(ei_hbm.at[1, pl.ds(base, _E_PER)], dstv, sem_in)
    cp3 = pltpu.async_copy(w_hbm.at[pl.ds(base, _E_PER)], wv, sem_in)
    cp4 = pltpu.async_copy(batch_hbm, batch_v, sem_in)
    cp5 = pltpu.async_copy(part_hbm.at[:, pl.ds(nbase, _NPT)], pb, sem_in)

    zeros = jnp.zeros((_L,), jnp.float32)

    def zb(i, carry):
        zv[pl.ds(i * _L, _L)] = zeros
        return carry

    lax.fori_loop(0, _CO_PER // _L, zb, 0)
    cp1.wait()
    cp2.wait()
    cp3.wait()
    cp4.wait()
    cp5.wait()

    def pk(k, carry):
        col = pl.ds(k * _L, _L)
        acc = pb[0, col]
        for p in range(1, _NW):
            acc = acc + pb[p, col]
        dinv_t[col] = _rsqrt16(acc + 1.0)
        return carry

    lax.fori_loop(0, _NPT // _L, pk, 0)
    pltpu.sync_copy(dinv_t, dinv_sp.at[pl.ds(nbase, _NPT)])
    pltpu.sync_copy(zv, coeff_sp.at[pl.ds(s * _CO_PER, _CO_PER)])
    plsc.subcore_barrier()
    pltpu.sync_copy(dinv_sp, dinv_v)

    def fire(r):
        return pltpu.async_copy(
            val_st.at[r], coeff_sp.at[idx_st.at[r]], sem_sc, add=True
        )

    def drain(r):
        pltpu.make_async_copy(
            val_st.at[r], coeff_sp.at[idx_st.at[r]], sem_sc
        ).wait()

    def edge_slot(r, u, off):
        sv = srcv[pl.ds(off, _L)]
        dv = dstv[pl.ds(off, _L)]
        w16 = wv[pl.ds(off, _L)]
        dis = plsc.load_gather(dinv_v, [sv])
        did = plsc.load_gather(dinv_v, [dv])
        g = plsc.load_gather(batch_v, [dv])
        idx_st[r, pl.ds(u * _L, _L)] = g * _N + sv
        val_st[r, pl.ds(u * _L, _L)] = dis * w16 * did

    def rb(r, carry):
        for u in range(8):
            edge_slot(r, u, r * 128 + u * _L)
        fire(r)

        @pl.when(r >= _PIPE)
        def _():
            drain(r - _PIPE)

        return carry

    lax.fori_loop(0, 78, rb, 0)

    edge_slot(78, 0, 624 * _L)
    iota16 = lax.iota(jnp.int32, _L)
    for j in range(20):
        q = _EV + j
        r, u = q // 8, q % 8
        v = wid + 32 * j
        n0 = jnp.minimum(v, _EV - 1) * _L
        g = batch_v[pl.ds(n0, _L)]
        y = dinv_v[pl.ds(n0, _L)]
        val = y * y
        if j == 19:
            val = val * jnp.where(v < _EV, 1.0, 0.0)
        idx_st[r, pl.ds(u * _L, _L)] = g * _N + n0 + iota16
        val_st[r, pl.ds(u * _L, _L)] = val
    for q in range(_EV + 20, _SLOTS):
        idx_st[q // 8, pl.ds((q % 8) * _L, _L)] = jnp.zeros((_L,), jnp.int32)
        val_st[q // 8, pl.ds((q % 8) * _L, _L)] = zeros
    for r in (78, 79, 80):
        fire(r)
    for r in range(78 - _PIPE, _ROWS):
        drain(r)

    plsc.subcore_barrier()
    pltpu.sync_copy(coeff_sp.at[pl.ds(s * _CO_PER, _CO_PER)], out_hbm.at[c, s])


_coeff_kernel = pl.kernel(
    _coeff_body,
    out_type=jax.ShapeDtypeStruct((_NC, _G, _N), jnp.float32),
    mesh=_mesh,
    compiler_params=_sc_params,
    scratch_types=[
        pltpu.VMEM((_E_PER,), jnp.int32),
        pltpu.VMEM((_E_PER,), jnp.int32),
        pltpu.VMEM((_E_PER,), jnp.float32),
        pltpu.VMEM((_NW, _NPT), jnp.float32),
        pltpu.VMEM((_NPT,), jnp.float32),
        pltpu.VMEM((_NP,), jnp.float32),
        pltpu.VMEM((_N,), jnp.int32),
        pltpu.VMEM((_ROWS, 128), jnp.int32),
        pltpu.VMEM((_ROWS, 128), jnp.float32),
        pltpu.VMEM((_CO_PER,), jnp.float32),
        pltpu.VMEM_SHARED((_NP,), jnp.float32),
        pltpu.VMEM_SHARED((_COEFF,), jnp.float32),
        pltpu.SemaphoreType.DMA,
        pltpu.SemaphoreType.DMA,
    ],
)


def _pool_body(c_ref, x_ref, w_ref, bb_ref, batch_ref, out_ref):
    bv = jnp.broadcast_to(jnp.reshape(batch_ref[...], (1, _N)), (_G, _N))
    gi = lax.broadcasted_iota(jnp.int32, (_G, _N), 0)
    cnt = jnp.sum((bv == gi).astype(jnp.float32), axis=1, keepdims=True)
    pos = cnt > 0.0
    invc = jnp.where(pos, 1.0 / jnp.maximum(cnt, 1.0), 0.0)
    bmask = jnp.where(pos, 1.0, 0.0)
    c2 = c_ref[0] + c_ref[1]
    s = lax.dot_general(c2, x_ref[...], (((1,), (0,)), ((), ())),
                        preferred_element_type=jnp.float32)
    p = lax.dot_general(s, w_ref[...], (((1,), (0,)), ((), ())),
                        preferred_element_type=jnp.float32)
    out_ref[...] = p * invc + bb_ref[...] * bmask


_pool_kernel = pl.pallas_call(
    _pool_body,
    out_shape=jax.ShapeDtypeStruct((_G, _C), jnp.float32),
)


def kernel(x, edge_index, edge_weights, batch, W, b):
    ei = edge_index.astype(jnp.int32)
    batch32 = batch.astype(jnp.int32)
    w = edge_weights.astype(jnp.float32)

    part = _deg_kernel(ei, w)
    coeffp = _coeff_kernel(ei, w, part, batch32)
    return _pool_kernel(coeffp, x, W, b.reshape(1, _C), batch32)


# speedup vs baseline: 137.1812x; 1.0117x over previous
import jax
import jax.numpy as jnp
from jax import lax
from jax.experimental import pallas as pl
from jax.experimental.pallas import tpu as pltpu
from jax.experimental.pallas import tpu_sc as plsc

_N = 10000
_NP = 10240
_E = 320000
_D = 128
_C = 16
_G = 16

_NC = 2
_NS = 16
_NW = _NC * _NS
_L = 16

_EB = 9984
_ET = _EB + 128
_EREM = _NW * _EB
_EV = 625
_NPT = _NP // _NS
_SLOTS = 656
_ROWS = _SLOTS // 8
_COEFF = _G * _NP
_CO_PER = _COEFF // _NS
_PIPE = 4

_mesh = plsc.VectorSubcoreMesh(
    core_axis_name="c", subcore_axis_name="s", num_cores=_NC, num_subcores=_NS
)
_sc_params = pltpu.CompilerParams(
    needs_layout_passes=False, use_tc_tiling_on_sc=True
)


def _rsqrt16(x):
    magic = jnp.full((_L,), 0x5F3759DF, jnp.int32)
    y = plsc.bitcast(magic - (plsc.bitcast(x, jnp.int32) >> 1), jnp.float32)
    for _ in range(3):
        y = y * (1.5 - 0.5 * x * y * y)
    return y


def _deg_body(ei_hbm, w_hbm, part_hbm, dstv, wv, degv, sem):
    c = lax.axis_index("c")
    s = lax.axis_index("s")
    wid = c * _NS + s
    base = wid * _EB
    rem = _EREM + jnp.minimum(wid, _NW // 8 - 1) * 128
    cp1 = pltpu.async_copy(ei_hbm.at[1, pl.ds(base, _EB)],
                           dstv.at[pl.ds(0, _EB)], sem)
    cp2 = pltpu.async_copy(w_hbm.at[pl.ds(base, _EB)],
                           wv.at[pl.ds(0, _EB)], sem)
    cp3 = pltpu.async_copy(ei_hbm.at[1, pl.ds(rem, 128)],
                           dstv.at[pl.ds(_EB, 128)], sem)
    cp4 = pltpu.async_copy(w_hbm.at[pl.ds(rem, 128)],
                           wv.at[pl.ds(_EB, 128)], sem)

    zeros = jnp.zeros((_L,), jnp.float32)

    def zb(i, carry):
        degv[pl.ds(i * _L, _L)] = zeros
        return carry

    lax.fori_loop(0, _NP // _L, zb, 0)
    cp1.wait()
    cp2.wait()
    cp3.wait()
    cp4.wait()

    tmask = jnp.where(wid < _NW // 8, 1.0, 0.0)
    for u in range(128 // _L):
        off = _EB + u * _L
        wv[pl.ds(off, _L)] = wv[pl.ds(off, _L)] * tmask

    def eb(i, carry):
        d = dstv[pl.ds(i * _L, _L)]
        w16 = wv[pl.ds(i * _L, _L)]
        plsc.addupdate_scatter(degv, [d], w16)
        return carry

    lax.fori_loop(0, _ET // _L, eb, 0)
    pltpu.sync_copy(degv, part_hbm.at[pl.ds(wid * _NP, _NP)])


_deg_kernel = pl.kernel(
    _deg_body,
    out_type=jax.ShapeDtypeStruct((_NW * _NP,), jnp.float32),
    mesh=_mesh,
    compiler_params=_sc_params,
    scratch_types=[
        pltpu.VMEM((_ET,), jnp.int32),
        pltpu.VMEM((_ET,), jnp.float32),
        pltpu.VMEM((_NP,), jnp.float32),
        pltpu.SemaphoreType.DMA,
    ],
)


def _coeff_body(ei_hbm, w_hbm, part_hbm, batch_hbm, out_hbm,
                srcv, dstv, wv, pb, dinv_t, dinv_v, batch_v, idx_st, val_st,
                zv, dinv_sp, coeff_sp, sem_in, sem_sc):
    c = lax.axis_index("c")
    s = lax.axis_index("s")
    wid = c * _NS + s
    base = wid * _EB
    rem = _EREM + jnp.minimum(wid, _NW // 8 - 1) * 128
    nbase = s * _NPT

    cps = [
        pltpu.async_copy(ei_hbm.at[0, pl.ds(base, _EB)],
                         srcv.at[pl.ds(0, _EB)], sem_in),
        pltpu.async_copy(ei_hbm.at[1, pl.ds(base, _EB)],
                         dstv.at[pl.ds(0, _EB)], sem_in),
        pltpu.async_copy(w_hbm.at[pl.ds(base, _EB)],
                         wv.at[pl.ds(0, _EB)], sem_in),
        pltpu.async_copy(ei_hbm.at[0, pl.ds(rem, 128)],
                         srcv.at[pl.ds(_EB, 128)], sem_in),
        pltpu.async_copy(ei_hbm.at[1, pl.ds(rem, 128)],
                         dstv.at[pl.ds(_EB, 128)], sem_in),
        pltpu.async_copy(w_hbm.at[pl.ds(rem, 128)],
                         wv.at[pl.ds(_EB, 128)], sem_in),
        pltpu.async_copy(batch_hbm, batch_v, sem_in),
    ]
    for p in range(_NW):
        cps.append(pltpu.async_copy(
            part_hbm.at[pl.ds(p * _NP + nbase, _NPT)], pb.at[p], sem_in))

    zeros = jnp.zeros((_L,), jnp.float32)

    def zb(i, carry):
        zv[pl.ds(i * _L, _L)] = zeros
        return carry

    lax.fori_loop(0, _CO_PER // _L, zb, 0)
    for cp in cps:
        cp.wait()

    tmask = jnp.where(wid < _NW // 8, 1.0, 0.0)
    for u in range(128 // _L):
        off = _EB + u * _L
        wv[pl.ds(off, _L)] = wv[pl.ds(off, _L)] * tmask

    def pk(k, carry):
        col = pl.ds(k * _L, _L)
        acc = pb[0, col]
        for p in range(1, _NW):
            acc = acc + pb[p, col]
        dinv_t[col] = _rsqrt16(acc + 1.0)
        return carry

    lax.fori_loop(0, _NPT // _L, pk, 0)
    pltpu.sync_copy(dinv_t, dinv_sp.at[pl.ds(nbase, _NPT)])
    pltpu.sync_copy(zv, coeff_sp.at[pl.ds(s * _CO_PER, _CO_PER)])
    plsc.subcore_barrier()
    pltpu.sync_copy(dinv_sp, dinv_v)

    def fire(r):
        return pltpu.async_copy(
            val_st.at[r], coeff_sp.at[idx_st.at[r]], sem_sc, add=True
        )

    def drain(r):
        pltpu.make_async_copy(
            val_st.at[r], coeff_sp.at[idx_st.at[r]], sem_sc
        ).wait()

    def edge_slot(r, u, off):
        sv = srcv[pl.ds(off, _L)]
        dv = dstv[pl.ds(off, _L)]
        w16 = wv[pl.ds(off, _L)]
        dis = plsc.load_gather(dinv_v, [sv])
        did = plsc.load_gather(dinv_v, [dv])
        g = plsc.load_gather(batch_v, [dv])
        idx_st[r, pl.ds(u * _L, _L)] = g * _NP + sv
        val_st[r, pl.ds(u * _L, _L)] = dis * w16 * did

    def rb(r, carry):
        for u in range(8):
            edge_slot(r, u, r * 128 + u * _L)
        fire(r)

        @pl.when(r >= _PIPE)
        def _():
            drain(r - _PIPE)

        return carry

    lax.fori_loop(0, 78, rb, 0)

    for u in range(8):
        edge_slot(78, u, _EB + u * _L)
    iota16 = lax.iota(jnp.int32, _L)
    for j in range(20):
        q = 632 + j
        r, u = q // 8, q % 8
        v = wid + 32 * j
        n0 = jnp.minimum(v, _EV - 1) * _L
        g = batch_v[pl.ds(n0, _L)]
        y = dinv_v[pl.ds(n0, _L)]
        val = y * y
        if j == 19:
            val = val * jnp.where(v < _EV, 1.0, 0.0)
        idx_st[r, pl.ds(u * _L, _L)] = g * _NP + n0 + iota16
        val_st[r, pl.ds(u * _L, _L)] = val
    for q in range(652, _SLOTS):
        idx_st[q // 8, pl.ds((q % 8) * _L, _L)] = jnp.zeros((_L,), jnp.int32)
        val_st[q // 8, pl.ds((q % 8) * _L, _L)] = zeros
    for r in (78, 79, 80, 81):
        fire(r)
    for r in range(78 - _PIPE, _ROWS):
        drain(r)

    plsc.subcore_barrier()
    wo = []
    for g in range(_G):
        wo.append(pltpu.async_copy(
            coeff_sp.at[pl.ds(g * _NP + nbase, _NPT)],
            out_hbm.at[c, g, pl.ds(nbase, _NPT)], sem_in))
    for cp in wo:
        cp.wait()


_coeff_kernel = pl.kernel(
    _coeff_body,
    out_type=jax.ShapeDtypeStruct((_NC, _G, _NP), jnp.float32),
    mesh=_mesh,
    compiler_params=_sc_params,
    scratch_types=[
        pltpu.VMEM((_ET,), jnp.int32),
        pltpu.VMEM((_ET,), jnp.int32),
        pltpu.VMEM((_ET,), jnp.float32),
        pltpu.VMEM((_NW, _NPT), jnp.float32),
        pltpu.VMEM((_NPT,), jnp.float32),
        pltpu.VMEM((_NP,), jnp.float32),
        pltpu.VMEM((_N,), jnp.int32),
        pltpu.VMEM((_ROWS, 128), jnp.int32),
        pltpu.VMEM((_ROWS, 128), jnp.float32),
        pltpu.VMEM((_CO_PER,), jnp.float32),
        pltpu.VMEM_SHARED((_NP,), jnp.float32),
        pltpu.VMEM_SHARED((_COEFF,), jnp.float32),
        pltpu.SemaphoreType.DMA,
        pltpu.SemaphoreType.DMA,
    ],
)


def _pool_body(c_ref, x_ref, w_ref, bb_ref, batch_ref, out_ref):
    bv = jnp.broadcast_to(jnp.reshape(batch_ref[...], (1, _N)), (_G, _N))
    gi = lax.broadcasted_iota(jnp.int32, (_G, _N), 0)
    cnt = jnp.sum((bv == gi).astype(jnp.float32), axis=1, keepdims=True)
    pos = cnt > 0.0
    invc = jnp.where(pos, 1.0 / jnp.maximum(cnt, 1.0), 0.0)
    bmask = jnp.where(pos, 1.0, 0.0)
    c2 = (c_ref[0] + c_ref[1])[:, :_N]
    s = lax.dot_general(c2, x_ref[...], (((1,), (0,)), ((), ())),
                        preferred_element_type=jnp.float32)
    p = lax.dot_general(s, w_ref[...], (((1,), (0,)), ((), ())),
                        preferred_element_type=jnp.float32)
    out_ref[...] = p * invc + bb_ref[...] * bmask


_pool_kernel = pl.pallas_call(
    _pool_body,
    out_shape=jax.ShapeDtypeStruct((_G, _C), jnp.float32),
)


def kernel(x, edge_index, edge_weights, batch, W, b):
    ei = edge_index.astype(jnp.int32)
    batch32 = batch.astype(jnp.int32)
    w = edge_weights.astype(jnp.float32)

    part = _deg_kernel(ei, w)
    coeffp = _coeff_kernel(ei, w, part, batch32)
    return _pool_kernel(coeffp, x, W, b.reshape(1, _C), batch32)


# speedup vs baseline: 137.4777x; 1.0022x over previous
import jax
import jax.numpy as jnp
from jax import lax
from jax.experimental import pallas as pl
from jax.experimental.pallas import tpu as pltpu
from jax.experimental.pallas import tpu_sc as plsc

_N = 10000
_NP = 10240
_E = 320000
_D = 128
_C = 16
_G = 16

_NC = 2
_NS = 16
_NW = _NC * _NS
_L = 16

_EA = 19968
_EAT = _EA + 128
_EREM = _NS * _EA
_EB = _EA // 2
_EV = 625
_NPT = _NP // _NS
_SLOTS = 648
_ROWS = _SLOTS // 8
_COEFF = _G * _NP
_PIPE = 4

_mesh = plsc.VectorSubcoreMesh(
    core_axis_name="c", subcore_axis_name="s", num_cores=_NC, num_subcores=_NS
)
_sc_params = pltpu.CompilerParams(
    needs_layout_passes=False, use_tc_tiling_on_sc=True
)


def _rsqrt16(x):
    magic = jnp.full((_L,), 0x5F3759DF, jnp.int32)
    y = plsc.bitcast(magic - (plsc.bitcast(x, jnp.int32) >> 1), jnp.float32)
    for _ in range(3):
        y = y * (1.5 - 0.5 * x * y * y)
    return y


def _gcn_body(ei_hbm, w_hbm, batch_hbm, out_hbm,
              dstv, wv, srcv, pb, dinv_t, dinv_v, batch_v, idx_st, val_st,
              degv, dinv_sp, coeff_sp, sem_in, sem_sc):
    c = lax.axis_index("c")
    s = lax.axis_index("s")
    wid = c * _NS + s
    abase = s * _EA
    rem = _EREM + jnp.minimum(s, _NS // 4 - 1) * 128
    nbase = s * _NPT
    bbase = c * _EB
    rbase = _EA + c * 64

    cps = [
        pltpu.async_copy(ei_hbm.at[1, pl.ds(abase, _EA)],
                         dstv.at[pl.ds(0, _EA)], sem_in),
        pltpu.async_copy(w_hbm.at[pl.ds(abase, _EA)],
                         wv.at[pl.ds(0, _EA)], sem_in),
        pltpu.async_copy(ei_hbm.at[1, pl.ds(rem, 128)],
                         dstv.at[pl.ds(_EA, 128)], sem_in),
        pltpu.async_copy(w_hbm.at[pl.ds(rem, 128)],
                         wv.at[pl.ds(_EA, 128)], sem_in),
        pltpu.async_copy(ei_hbm.at[0, pl.ds(abase + bbase, _EB)],
                         srcv.at[pl.ds(0, _EB)], sem_in),
        pltpu.async_copy(ei_hbm.at[0, pl.ds(rem, 128)],
                         srcv.at[pl.ds(_EB, 128)], sem_in),
        pltpu.async_copy(batch_hbm, batch_v, sem_in),
    ]

    zeros = jnp.zeros((_L,), jnp.float32)

    def zb(i, carry):
        degv[pl.ds(i * _L, _L)] = zeros
        return carry

    lax.fori_loop(0, _NP // _L, zb, 0)
    for cp in cps:
        cp.wait()

    tmask = jnp.where(s < _NS // 4, 1.0, 0.0)
    for u in range(128 // _L):
        off = _EA + u * _L
        wv[pl.ds(off, _L)] = wv[pl.ds(off, _L)] * tmask

    def eb(i, carry):
        d = dstv[pl.ds(i * _L, _L)]
        w16 = wv[pl.ds(i * _L, _L)]
        plsc.addupdate_scatter(degv, [d], w16)
        return carry

    lax.fori_loop(0, _EAT // _L, eb, 0)
    pltpu.sync_copy(degv, coeff_sp.at[pl.ds(s * _NP, _NP)])

    lax.fori_loop(0, _NP // _L, zb, 0)
    plsc.subcore_barrier()

    pbs = [
        pltpu.async_copy(coeff_sp.at[pl.ds(p * _NP + nbase, _NPT)],
                         pb.at[p], sem_in)
        for p in range(_NS)
    ]
    for cp in pbs:
        cp.wait()
    plsc.subcore_barrier()

    def pk(k, carry):
        col = pl.ds(k * _L, _L)
        acc = pb[0, col]
        for p in range(1, _NS):
            acc = acc + pb[p, col]
        dinv_t[col] = _rsqrt16(acc + 1.0)
        return carry

    lax.fori_loop(0, _NPT // _L, pk, 0)
    pltpu.sync_copy(dinv_t, dinv_sp.at[pl.ds(nbase, _NPT)])
    pltpu.sync_copy(degv, coeff_sp.at[pl.ds(s * _NP, _NP)])
    plsc.subcore_barrier()
    pltpu.sync_copy(dinv_sp, dinv_v)

    def fire(r):
        return pltpu.async_copy(
            val_st.at[r], coeff_sp.at[idx_st.at[r]], sem_sc, add=True
        )

    def drain(r):
        pltpu.make_async_copy(
            val_st.at[r], coeff_sp.at[idx_st.at[r]], sem_sc
        ).wait()

    def edge_slot(r, u, soff, dwoff):
        sv = srcv[pl.ds(soff, _L)]
        dv = dstv[pl.ds(dwoff, _L)]
        w16 = wv[pl.ds(dwoff, _L)]
        dis = plsc.load_gather(dinv_v, [sv])
        did = plsc.load_gather(dinv_v, [dv])
        g = plsc.load_gather(batch_v, [dv])
        idx_st[r, pl.ds(u * _L, _L)] = g * _NP + sv
        val_st[r, pl.ds(u * _L, _L)] = dis * w16 * did

    def rb(r, carry):
        for u in range(8):
            off = r * 128 + u * _L
            edge_slot(r, u, off, bbase + off)
        fire(r)

        @pl.when(r >= _PIPE)
        def _():
            drain(r - _PIPE)

        return carry

    lax.fori_loop(0, 78, rb, 0)

    for u in range(4):
        edge_slot(78, u, _EB + c * 64 + u * _L, rbase + u * _L)
    iota16 = lax.iota(jnp.int32, _L)
    for j in range(20):
        q = 628 + j
        r, u = q // 8, q % 8
        v = wid + 32 * j
        n0 = jnp.minimum(v, _EV - 1) * _L
        g = batch_v[pl.ds(n0, _L)]
        y = dinv_v[pl.ds(n0, _L)]
        val = y * y
        if j == 19:
            val = val * jnp.where(v < _EV, 1.0, 0.0)
        idx_st[r, pl.ds(u * _L, _L)] = g * _NP + n0 + iota16
        val_st[r, pl.ds(u * _L, _L)] = val
    for r in (78, 79, 80):
        fire(r)
    for r in range(78 - _PIPE, _ROWS):
        drain(r)

    plsc.subcore_barrier()
    wo = [
        pltpu.async_copy(coeff_sp.at[pl.ds(g * _NP + nbase, _NPT)],
                         out_hbm.at[c, g, pl.ds(nbase, _NPT)], sem_in)
        for g in range(_G)
    ]
    for cp in wo:
        cp.wait()


_gcn_kernel = pl.kernel(
    _gcn_body,
    out_type=jax.ShapeDtypeStruct((_NC, _G, _NP), jnp.float32),
    mesh=_mesh,
    compiler_params=_sc_params,
    scratch_types=[
        pltpu.VMEM((_EAT,), jnp.int32),
        pltpu.VMEM((_EAT,), jnp.float32),
        pltpu.VMEM((_EB + 128,), jnp.int32),
        pltpu.VMEM((_NS, _NPT), jnp.float32),
        pltpu.VMEM((_NPT,), jnp.float32),
        pltpu.VMEM((_NP,), jnp.float32),
        pltpu.VMEM((_N,), jnp.int32),
        pltpu.VMEM((_ROWS, 128), jnp.int32),
        pltpu.VMEM((_ROWS, 128), jnp.float32),
        pltpu.VMEM((_NP,), jnp.float32),
        pltpu.VMEM_SHARED((_NP,), jnp.float32),
        pltpu.VMEM_SHARED((_COEFF,), jnp.float32),
        pltpu.SemaphoreType.DMA,
        pltpu.SemaphoreType.DMA,
    ],
)


def _pool_body(c_ref, x_ref, w_ref, bb_ref, batch_ref, out_ref):
    bv = jnp.broadcast_to(jnp.reshape(batch_ref[...], (1, _N)), (_G, _N))
    gi = lax.broadcasted_iota(jnp.int32, (_G, _N), 0)
    cnt = jnp.sum((bv == gi).astype(jnp.float32), axis=1, keepdims=True)
    pos = cnt > 0.0
    invc = jnp.where(pos, 1.0 / jnp.maximum(cnt, 1.0), 0.0)
    bmask = jnp.where(pos, 1.0, 0.0)
    c2 = (c_ref[0] + c_ref[1])[:, :_N]
    s = lax.dot_general(c2, x_ref[...], (((1,), (0,)), ((), ())),
                        preferred_element_type=jnp.float32)
    p = lax.dot_general(s, w_ref[...], (((1,), (0,)), ((), ())),
                        preferred_element_type=jnp.float32)
    out_ref[...] = p * invc + bb_ref[...] * bmask


_pool_kernel = pl.pallas_call(
    _pool_body,
    out_shape=jax.ShapeDtypeStruct((_G, _C), jnp.float32),
)


def kernel(x, edge_index, edge_weights, batch, W, b):
    ei = edge_index.astype(jnp.int32)
    batch32 = batch.astype(jnp.int32)
    w = edge_weights.astype(jnp.float32)

    coeffp = _gcn_kernel(ei, w, batch32)
    return _pool_kernel(coeffp, x, W, b.reshape(1, _C), batch32)


# speedup vs baseline: 144.4569x; 1.0508x over previous
import jax
import jax.numpy as jnp
from jax import lax
from jax.experimental import pallas as pl
from jax.experimental.pallas import tpu as pltpu
from jax.experimental.pallas import tpu_sc as plsc

_N = 10000
_NP = 10240
_E = 320000
_D = 128
_C = 16
_G = 16

_NC = 2
_NS = 16
_NW = _NC * _NS
_L = 16

_EA = 19968
_EAT = _EA + 128
_EREM = _NS * _EA
_EB = _EA // 2
_EV = 625
_NPT = _NP // _NS
_SLOTS = 648
_ROWS = _SLOTS // 8
_COEFF = _G * _NP
_PIPE = 4

_mesh = plsc.VectorSubcoreMesh(
    core_axis_name="c", subcore_axis_name="s", num_cores=_NC, num_subcores=_NS
)
_sc_params = pltpu.CompilerParams(
    needs_layout_passes=False, use_tc_tiling_on_sc=True
)


def _rsqrt16(x):
    magic = jnp.full((_L,), 0x5F3759DF, jnp.int32)
    y = plsc.bitcast(magic - (plsc.bitcast(x, jnp.int32) >> 1), jnp.float32)
    for _ in range(3):
        y = y * (1.5 - 0.5 * x * y * y)
    return y


def _gcn_body(ei_hbm, w_hbm, batch_hbm, out_hbm,
              dstv, wv, srcv, pb, dinv_t, dinv_v, batch_v, idx_st, val_st,
              degv, dinv_sp, coeff_sp, sem_in, sem_sc):
    c = lax.axis_index("c")
    s = lax.axis_index("s")
    wid = c * _NS + s
    abase = s * _EA
    rem = _EREM + jnp.minimum(s, _NS // 4 - 1) * 128
    nbase = s * _NPT
    bbase = c * _EB
    rbase = _EA + c * 64

    cps = [
        pltpu.async_copy(ei_hbm.at[1, pl.ds(abase, _EA)],
                         dstv.at[pl.ds(0, _EA)], sem_in),
        pltpu.async_copy(w_hbm.at[pl.ds(abase, _EA)],
                         wv.at[pl.ds(0, _EA)], sem_in),
        pltpu.async_copy(ei_hbm.at[1, pl.ds(rem, 128)],
                         dstv.at[pl.ds(_EA, 128)], sem_in),
        pltpu.async_copy(w_hbm.at[pl.ds(rem, 128)],
                         wv.at[pl.ds(_EA, 128)], sem_in),
        pltpu.async_copy(ei_hbm.at[0, pl.ds(abase + bbase, _EB)],
                         srcv.at[pl.ds(0, _EB)], sem_in),
        pltpu.async_copy(ei_hbm.at[0, pl.ds(rem, 128)],
                         srcv.at[pl.ds(_EB, 128)], sem_in),
        pltpu.async_copy(batch_hbm, batch_v, sem_in),
    ]

    zeros = jnp.zeros((_L,), jnp.float32)

    def zb(i, carry):
        for u in range(8):
            degv[pl.ds(i * 128 + u * _L, _L)] = zeros
        return carry

    lax.fori_loop(0, _NP // 128, zb, 0)
    for cp in cps:
        cp.wait()

    tmask = jnp.where(s < _NS // 4, 1.0, 0.0)
    for u in range(128 // _L):
        off = _EA + u * _L
        wv[pl.ds(off, _L)] = wv[pl.ds(off, _L)] * tmask

    def eb(i, carry):
        for u in range(8):
            off = i * 128 + u * _L
            d = dstv[pl.ds(off, _L)]
            w16 = wv[pl.ds(off, _L)]
            plsc.addupdate_scatter(degv, [d], w16)
        return carry

    lax.fori_loop(0, _EAT // 128, eb, 0)
    pltpu.sync_copy(degv, coeff_sp.at[pl.ds(s * _NP, _NP)])

    lax.fori_loop(0, _NP // 128, zb, 0)
    plsc.subcore_barrier()

    pbs = [
        pltpu.async_copy(coeff_sp.at[pl.ds(p * _NP + nbase, _NPT)],
                         pb.at[p], sem_in)
        for p in range(_NS)
    ]
    for cp in pbs:
        cp.wait()
    plsc.subcore_barrier()

    def pk(k, carry):
        col = pl.ds(k * _L, _L)
        acc = pb[0, col]
        for p in range(1, _NS):
            acc = acc + pb[p, col]
        dinv_t[col] = _rsqrt16(acc + 1.0)
        return carry

    lax.fori_loop(0, _NPT // _L, pk, 0)
    pltpu.sync_copy(dinv_t, dinv_sp.at[pl.ds(nbase, _NPT)])
    pltpu.sync_copy(degv, coeff_sp.at[pl.ds(s * _NP, _NP)])
    plsc.subcore_barrier()
    pltpu.sync_copy(dinv_sp, dinv_v)

    def fire(r):
        return pltpu.async_copy(
            val_st.at[r], coeff_sp.at[idx_st.at[r]], sem_sc, add=True
        )

    def drain(r):
        pltpu.make_async_copy(
            val_st.at[r], coeff_sp.at[idx_st.at[r]], sem_sc
        ).wait()

    def edge_slot(r, u, soff, dwoff):
        sv = srcv[pl.ds(soff, _L)]
        dv = dstv[pl.ds(dwoff, _L)]
        w16 = wv[pl.ds(dwoff, _L)]
        dis = plsc.load_gather(dinv_v, [sv])
        did = plsc.load_gather(dinv_v, [dv])
        g = plsc.load_gather(batch_v, [dv])
        idx_st[r, pl.ds(u * _L, _L)] = g * _NP + sv
        val_st[r, pl.ds(u * _L, _L)] = dis * w16 * did

    def rb(r, carry):
        for u in range(8):
            off = r * 128 + u * _L
            edge_slot(r, u, off, bbase + off)
        fire(r)

        @pl.when(r >= _PIPE)
        def _():
            drain(r - _PIPE)

        return carry

    lax.fori_loop(0, 78, rb, 0)

    for u in range(4):
        edge_slot(78, u, _EB + c * 64 + u * _L, rbase + u * _L)
    iota16 = lax.iota(jnp.int32, _L)
    for j in range(20):
        q = 628 + j
        r, u = q // 8, q % 8
        v = wid + 32 * j
        n0 = jnp.minimum(v, _EV - 1) * _L
        g = batch_v[pl.ds(n0, _L)]
        y = dinv_v[pl.ds(n0, _L)]
        val = y * y
        if j == 19:
            val = val * jnp.where(v < _EV, 1.0, 0.0)
        idx_st[r, pl.ds(u * _L, _L)] = g * _NP + n0 + iota16
        val_st[r, pl.ds(u * _L, _L)] = val
    for r in (78, 79, 80):
        fire(r)
    for r in range(78 - _PIPE, _ROWS):
        drain(r)

    plsc.subcore_barrier()
    wo = [
        pltpu.async_copy(coeff_sp.at[pl.ds(g * _NP + nbase, _NPT)],
                         out_hbm.at[c, g, pl.ds(nbase, _NPT)], sem_in)
        for g in range(_G)
    ]
    for cp in wo:
        cp.wait()


_gcn_kernel = pl.kernel(
    _gcn_body,
    out_type=jax.ShapeDtypeStruct((_NC, _G, _NP), jnp.float32),
    mesh=_mesh,
    compiler_params=_sc_params,
    scratch_types=[
        pltpu.VMEM((_EAT,), jnp.int32),
        pltpu.VMEM((_EAT,), jnp.float32),
        pltpu.VMEM((_EB + 128,), jnp.int32),
        pltpu.VMEM((_NS, _NPT), jnp.float32),
        pltpu.VMEM((_NPT,), jnp.float32),
        pltpu.VMEM((_NP,), jnp.float32),
        pltpu.VMEM((_N,), jnp.int32),
        pltpu.VMEM((_ROWS, 128), jnp.int32),
        pltpu.VMEM((_ROWS, 128), jnp.float32),
        pltpu.VMEM((_NP,), jnp.float32),
        pltpu.VMEM_SHARED((_NP,), jnp.float32),
        pltpu.VMEM_SHARED((_COEFF,), jnp.float32),
        pltpu.SemaphoreType.DMA,
        pltpu.SemaphoreType.DMA,
    ],
)


def _pool_body(c_ref, x_ref, w_ref, bb_ref, batch_ref, out_ref):
    bv = jnp.broadcast_to(jnp.reshape(batch_ref[...], (1, _N)), (_G, _N))
    gi = lax.broadcasted_iota(jnp.int32, (_G, _N), 0)
    cnt = jnp.sum((bv == gi).astype(jnp.float32), axis=1, keepdims=True)
    pos = cnt > 0.0
    invc = jnp.where(pos, 1.0 / jnp.maximum(cnt, 1.0), 0.0)
    bmask = jnp.where(pos, 1.0, 0.0)
    c2 = (c_ref[0] + c_ref[1])[:, :_N]
    s = lax.dot_general(c2, x_ref[...], (((1,), (0,)), ((), ())),
                        preferred_element_type=jnp.float32)
    p = lax.dot_general(s, w_ref[...], (((1,), (0,)), ((), ())),
                        preferred_element_type=jnp.float32)
    out_ref[...] = p * invc + bb_ref[...] * bmask


_pool_kernel = pl.pallas_call(
    _pool_body,
    out_shape=jax.ShapeDtypeStruct((_G, _C), jnp.float32),
)


def kernel(x, edge_index, edge_weights, batch, W, b):
    ei = edge_index.astype(jnp.int32)
    batch32 = batch.astype(jnp.int32)
    w = edge_weights.astype(jnp.float32)

    coeffp = _gcn_kernel(ei, w, batch32)
    return _pool_kernel(coeffp, x, W, b.reshape(1, _C), batch32)


# speedup vs baseline: 153.6232x; 1.0635x over previous
import jax
import jax.numpy as jnp
from jax import lax
from jax.experimental import pallas as pl
from jax.experimental.pallas import tpu as pltpu
from jax.experimental.pallas import tpu_sc as plsc

_N = 10000
_NP = 10240
_E = 320000
_D = 128
_C = 16
_G = 16

_NC = 2
_NS = 16
_NW = _NC * _NS
_L = 16

_EA = 19968
_EAT = _EA + 128
_EREM = _NS * _EA
_EB = _EA // 2
_EV = 625
_NPT = _NP // _NS
_SLOTS = 648
_ROWS = _SLOTS // 8
_COEFF = _G * _NP
_PIPE = 4

_mesh = plsc.VectorSubcoreMesh(
    core_axis_name="c", subcore_axis_name="s", num_cores=_NC, num_subcores=_NS
)
_sc_params = pltpu.CompilerParams(
    needs_layout_passes=False, use_tc_tiling_on_sc=True
)


def _rsqrt16(x):
    magic = jnp.full((_L,), 0x5F3759DF, jnp.int32)
    y = plsc.bitcast(magic - (plsc.bitcast(x, jnp.int32) >> 1), jnp.float32)
    for _ in range(3):
        y = y * (1.5 - 0.5 * x * y * y)
    return y


def _gcn_body(ei_hbm, w_hbm, batch_hbm, out_hbm,
              dstv, wv, srcv, pb, dinv_t, dinv_v, batch_t, idx_st, val_st,
              degv, dinv_sp, coeff_sp, sem_in, sem_sc):
    c = lax.axis_index("c")
    s = lax.axis_index("s")
    wid = c * _NS + s
    abase = s * _EA
    rem = _EREM + jnp.minimum(s, _NS // 4 - 1) * 128
    nbase = s * _NPT
    bbase = c * _EB
    rbase = _EA + c * 64

    cps = [
        pltpu.async_copy(ei_hbm.at[1, pl.ds(abase, _EA)],
                         dstv.at[pl.ds(0, _EA)], sem_in),
        pltpu.async_copy(w_hbm.at[pl.ds(abase, _EA)],
                         wv.at[pl.ds(0, _EA)], sem_in),
        pltpu.async_copy(ei_hbm.at[1, pl.ds(rem, 128)],
                         dstv.at[pl.ds(_EA, 128)], sem_in),
        pltpu.async_copy(w_hbm.at[pl.ds(rem, 128)],
                         wv.at[pl.ds(_EA, 128)], sem_in),
        pltpu.async_copy(ei_hbm.at[0, pl.ds(abase + bbase, _EB)],
                         srcv.at[pl.ds(0, _EB)], sem_in),
        pltpu.async_copy(ei_hbm.at[0, pl.ds(rem, 128)],
                         srcv.at[pl.ds(_EB, 128)], sem_in),
        pltpu.async_copy(
            batch_hbm.at[pl.ds(jnp.minimum(nbase, _N - _NPT), _NPT)],
            batch_t, sem_in),
    ]
    bshift = nbase - jnp.minimum(nbase, _N - _NPT)

    zeros = jnp.zeros((_L,), jnp.float32)

    def zb(i, carry):
        for u in range(8):
            degv[pl.ds(i * 128 + u * _L, _L)] = zeros
        return carry

    lax.fori_loop(0, _NP // 128, zb, 0)
    for cp in cps:
        cp.wait()

    tmask = jnp.where(s < _NS // 4, 1.0, 0.0)
    for u in range(128 // _L):
        off = _EA + u * _L
        wv[pl.ds(off, _L)] = wv[pl.ds(off, _L)] * tmask

    def eb(i, carry):
        for u in range(8):
            off = i * 128 + u * _L
            d = dstv[pl.ds(off, _L)]
            w16 = wv[pl.ds(off, _L)]
            plsc.addupdate_scatter(degv, [d], w16)
        return carry

    lax.fori_loop(0, _EAT // 128, eb, 0)
    pltpu.sync_copy(degv, coeff_sp.at[pl.ds(s * _NP, _NP)])

    lax.fori_loop(0, _NP // 128, zb, 0)
    plsc.subcore_barrier()

    pbs = [
        pltpu.async_copy(coeff_sp.at[pl.ds(p * _NP + nbase, _NPT)],
                         pb.at[p], sem_in)
        for p in range(_NS)
    ]
    for cp in pbs:
        cp.wait()
    plsc.subcore_barrier()

    def pk(k, carry):
        col = pl.ds(k * _L, _L)
        acc = pb[0, col]
        for p in range(1, _NS):
            acc = acc + pb[p, col]
        y = plsc.bitcast(_rsqrt16(acc + 1.0), jnp.int32)
        b16 = batch_t[pl.ds(jnp.minimum(k * _L + bshift, _NPT - _L), _L)]
        dinv_t[col] = plsc.bitcast((y & ~15) | b16, jnp.float32)
        return carry

    lax.fori_loop(0, _NPT // _L, pk, 0)
    pltpu.sync_copy(dinv_t, dinv_sp.at[pl.ds(nbase, _NPT)])
    pltpu.sync_copy(degv, coeff_sp.at[pl.ds(s * _NP, _NP)])
    plsc.subcore_barrier()
    pltpu.sync_copy(dinv_sp, dinv_v)

    def fire(r):
        return pltpu.async_copy(
            val_st.at[r], coeff_sp.at[idx_st.at[r]], sem_sc, add=True
        )

    def drain(r):
        pltpu.make_async_copy(
            val_st.at[r], coeff_sp.at[idx_st.at[r]], sem_sc
        ).wait()

    def edge_slot(r, u, soff, dwoff):
        sv = srcv[pl.ds(soff, _L)]
        dv = dstv[pl.ds(dwoff, _L)]
        w16 = wv[pl.ds(dwoff, _L)]
        ps = plsc.bitcast(plsc.load_gather(dinv_v, [sv]), jnp.int32)
        pd = plsc.bitcast(plsc.load_gather(dinv_v, [dv]), jnp.int32)
        dis = plsc.bitcast(ps & ~15, jnp.float32)
        did = plsc.bitcast(pd & ~15, jnp.float32)
        idx_st[r, pl.ds(u * _L, _L)] = (pd & 15) * _NP + sv
        val_st[r, pl.ds(u * _L, _L)] = dis * w16 * did

    def rb(r, carry):
        for u in range(8):
            off = r * 128 + u * _L
            edge_slot(r, u, off, bbase + off)
        fire(r)

        @pl.when(r >= _PIPE)
        def _():
            drain(r - _PIPE)

        return carry

    lax.fori_loop(0, 78, rb, 0)

    for u in range(4):
        edge_slot(78, u, _EB + c * 64 + u * _L, rbase + u * _L)
    iota16 = lax.iota(jnp.int32, _L)
    for j in range(20):
        q = 628 + j
        r, u = q // 8, q % 8
        v = wid + 32 * j
        n0 = jnp.minimum(v, _EV - 1) * _L
        pk16 = plsc.bitcast(dinv_v[pl.ds(n0, _L)], jnp.int32)
        y = plsc.bitcast(pk16 & ~15, jnp.float32)
        val = y * y
        if j == 19:
            val = val * jnp.where(v < _EV, 1.0, 0.0)
        idx_st[r, pl.ds(u * _L, _L)] = (pk16 & 15) * _NP + n0 + iota16
        val_st[r, pl.ds(u * _L, _L)] = val
    for r in (78, 79, 80):
        fire(r)
    for r in range(78 - _PIPE, _ROWS):
        drain(r)

    plsc.subcore_barrier()
    wo = [
        pltpu.async_copy(coeff_sp.at[pl.ds(g * _NP + nbase, _NPT)],
                         out_hbm.at[c, g, pl.ds(nbase, _NPT)], sem_in)
        for g in range(_G)
    ]
    for cp in wo:
        cp.wait()


_gcn_kernel = pl.kernel(
    _gcn_body,
    out_type=jax.ShapeDtypeStruct((_NC, _G, _NP), jnp.float32),
    mesh=_mesh,
    compiler_params=_sc_params,
    scratch_types=[
        pltpu.VMEM((_EAT,), jnp.int32),
        pltpu.VMEM((_EAT,), jnp.float32),
        pltpu.VMEM((_EB + 128,), jnp.int32),
        pltpu.VMEM((_NS, _NPT), jnp.float32),
        pltpu.VMEM((_NPT,), jnp.float32),
        pltpu.VMEM((_NP,), jnp.float32),
        pltpu.VMEM((_NPT,), jnp.int32),
        pltpu.VMEM((_ROWS, 128), jnp.int32),
        pltpu.VMEM((_ROWS, 128), jnp.float32),
        pltpu.VMEM((_NP,), jnp.float32),
        pltpu.VMEM_SHARED((_NP,), jnp.float32),
        pltpu.VMEM_SHARED((_COEFF,), jnp.float32),
        pltpu.SemaphoreType.DMA,
        pltpu.SemaphoreType.DMA,
    ],
)


def _pool_body(c_ref, x_ref, w_ref, bb_ref, batch_ref, out_ref):
    bv = jnp.broadcast_to(jnp.reshape(batch_ref[...], (1, _N)), (_G, _N))
    gi = lax.broadcasted_iota(jnp.int32, (_G, _N), 0)
    cnt = jnp.sum((bv == gi).astype(jnp.float32), axis=1, keepdims=True)
    pos = cnt > 0.0
    invc = jnp.where(pos, 1.0 / jnp.maximum(cnt, 1.0), 0.0)
    bmask = jnp.where(pos, 1.0, 0.0)
    c2 = (c_ref[0] + c_ref[1])[:, :_N]
    s = lax.dot_general(c2, x_ref[...], (((1,), (0,)), ((), ())),
                        preferred_element_type=jnp.float32)
    p = lax.dot_general(s, w_ref[...], (((1,), (0,)), ((), ())),
                        preferred_element_type=jnp.float32)
    out_ref[...] = p * invc + bb_ref[...] * bmask


_pool_kernel = pl.pallas_call(
    _pool_body,
    out_shape=jax.ShapeDtypeStruct((_G, _C), jnp.float32),
)


def kernel(x, edge_index, edge_weights, batch, W, b):
    ei = edge_index.astype(jnp.int32)
    batch32 = batch.astype(jnp.int32)
    w = edge_weights.astype(jnp.float32)

    coeffp = _gcn_kernel(ei, w, batch32)
    return _pool_kernel(coeffp, x, W, b.reshape(1, _C), batch32)


# speedup vs baseline: 153.9209x; 1.0019x over previous
import jax
import jax.numpy as jnp
from jax import lax
from jax.experimental import pallas as pl
from jax.experimental.pallas import tpu as pltpu
from jax.experimental.pallas import tpu_sc as plsc

_N = 10000
_NP = 10240
_E = 320000
_D = 128
_C = 16
_G = 16

_NC = 2
_NS = 16
_NW = _NC * _NS
_L = 16

_EA = 19968
_EAT = _EA + 128
_EREM = _NS * _EA
_EB = _EA // 2
_EV = 625
_NPT = _NP // _NS
_SLOTS = 648
_ROWS = _SLOTS // 8
_COEFF = _G * _NP
_PIPE = 8

_mesh = plsc.VectorSubcoreMesh(
    core_axis_name="c", subcore_axis_name="s", num_cores=_NC, num_subcores=_NS
)
_sc_params = pltpu.CompilerParams(
    needs_layout_passes=False, use_tc_tiling_on_sc=True
)


def _rsqrt16(x):
    magic = jnp.full((_L,), 0x5F3759DF, jnp.int32)
    y = plsc.bitcast(magic - (plsc.bitcast(x, jnp.int32) >> 1), jnp.float32)
    for _ in range(3):
        y = y * (1.5 - 0.5 * x * y * y)
    return y


def _gcn_body(ei_hbm, w_hbm, batch_hbm, out_hbm,
              dstv, wv, srcv, pb, dinv_t, dinv_v, batch_t, idx_st, val_st,
              degv, dinv_sp, coeff_sp, sem_in, sem_sc):
    c = lax.axis_index("c")
    s = lax.axis_index("s")
    wid = c * _NS + s
    abase = s * _EA
    rem = _EREM + jnp.minimum(s, _NS // 4 - 1) * 128
    nbase = s * _NPT
    bbase = c * _EB
    rbase = _EA + c * 64

    cps = [
        pltpu.async_copy(ei_hbm.at[1, pl.ds(abase, _EA)],
                         dstv.at[pl.ds(0, _EA)], sem_in),
        pltpu.async_copy(w_hbm.at[pl.ds(abase, _EA)],
                         wv.at[pl.ds(0, _EA)], sem_in),
        pltpu.async_copy(ei_hbm.at[1, pl.ds(rem, 128)],
                         dstv.at[pl.ds(_EA, 128)], sem_in),
        pltpu.async_copy(w_hbm.at[pl.ds(rem, 128)],
                         wv.at[pl.ds(_EA, 128)], sem_in),
        pltpu.async_copy(ei_hbm.at[0, pl.ds(abase + bbase, _EB)],
                         srcv.at[pl.ds(0, _EB)], sem_in),
        pltpu.async_copy(ei_hbm.at[0, pl.ds(rem, 128)],
                         srcv.at[pl.ds(_EB, 128)], sem_in),
        pltpu.async_copy(
            batch_hbm.at[pl.ds(jnp.minimum(nbase, _N - _NPT), _NPT)],
            batch_t, sem_in),
    ]
    bshift = nbase - jnp.minimum(nbase, _N - _NPT)

    zeros = jnp.zeros((_L,), jnp.float32)

    def zb(i, carry):
        for u in range(8):
            degv[pl.ds(i * 128 + u * _L, _L)] = zeros
        return carry

    lax.fori_loop(0, _NP // 128, zb, 0)
    for cp in cps[:4]:
        cp.wait()

    tmask = jnp.where(s < _NS // 4, 1.0, 0.0)
    for u in range(128 // _L):
        off = _EA + u * _L
        wv[pl.ds(off, _L)] = wv[pl.ds(off, _L)] * tmask

    def eb(i, carry):
        for u in range(8):
            off = i * 128 + u * _L
            d = dstv[pl.ds(off, _L)]
            w16 = wv[pl.ds(off, _L)]
            plsc.addupdate_scatter(degv, [d], w16)
        return carry

    lax.fori_loop(0, _EAT // 128, eb, 0)
    pltpu.sync_copy(degv, coeff_sp.at[pl.ds(s * _NP, _NP)])

    lax.fori_loop(0, _NP // 128, zb, 0)
    plsc.subcore_barrier()

    pbs = [
        pltpu.async_copy(coeff_sp.at[pl.ds(p * _NP + nbase, _NPT)],
                         pb.at[p], sem_in)
        for p in range(_NS)
    ]
    for cp in pbs:
        cp.wait()
    for cp in cps[4:]:
        cp.wait()
    plsc.subcore_barrier()

    def pk(k, carry):
        col = pl.ds(k * _L, _L)
        acc = pb[0, col]
        for p in range(1, _NS):
            acc = acc + pb[p, col]
        y = plsc.bitcast(_rsqrt16(acc + 1.0), jnp.int32)
        b16 = batch_t[pl.ds(jnp.minimum(k * _L + bshift, _NPT - _L), _L)]
        dinv_t[col] = plsc.bitcast((y & ~15) | b16, jnp.float32)
        return carry

    lax.fori_loop(0, _NPT // _L, pk, 0)
    pltpu.sync_copy(dinv_t, dinv_sp.at[pl.ds(nbase, _NPT)])
    pltpu.sync_copy(degv, coeff_sp.at[pl.ds(s * _NP, _NP)])
    plsc.subcore_barrier()
    pltpu.sync_copy(dinv_sp, dinv_v)

    def fire(r):
        return pltpu.async_copy(
            val_st.at[r], coeff_sp.at[idx_st.at[r]], sem_sc, add=True
        )

    def drain(r):
        pltpu.make_async_copy(
            val_st.at[r], coeff_sp.at[idx_st.at[r]], sem_sc
        ).wait()

    def edge_slot(r, u, soff, dwoff):
        sv = srcv[pl.ds(soff, _L)]
        dv = dstv[pl.ds(dwoff, _L)]
        w16 = wv[pl.ds(dwoff, _L)]
        ps = plsc.bitcast(plsc.load_gather(dinv_v, [sv]), jnp.int32)
        pd = plsc.bitcast(plsc.load_gather(dinv_v, [dv]), jnp.int32)
        dis = plsc.bitcast(ps & ~15, jnp.float32)
        did = plsc.bitcast(pd & ~15, jnp.float32)
        idx_st[r, pl.ds(u * _L, _L)] = (pd & 15) * _NP + sv
        val_st[r, pl.ds(u * _L, _L)] = dis * w16 * did

    def rb(r, carry):
        for u in range(8):
            off = r * 128 + u * _L
            edge_slot(r, u, off, bbase + off)
        fire(r)

        @pl.when(r >= _PIPE)
        def _():
            drain(r - _PIPE)

        return carry

    lax.fori_loop(0, 78, rb, 0)

    for u in range(4):
        edge_slot(78, u, _EB + c * 64 + u * _L, rbase + u * _L)
    iota16 = lax.iota(jnp.int32, _L)
    for j in range(20):
        q = 628 + j
        r, u = q // 8, q % 8
        v = wid + 32 * j
        n0 = jnp.minimum(v, _EV - 1) * _L
        pk16 = plsc.bitcast(dinv_v[pl.ds(n0, _L)], jnp.int32)
        y = plsc.bitcast(pk16 & ~15, jnp.float32)
        val = y * y
        if j == 19:
            val = val * jnp.where(v < _EV, 1.0, 0.0)
        idx_st[r, pl.ds(u * _L, _L)] = (pk16 & 15) * _NP + n0 + iota16
        val_st[r, pl.ds(u * _L, _L)] = val
    for r in (78, 79, 80):
        fire(r)
    for r in range(78 - _PIPE, _ROWS):
        drain(r)

    plsc.subcore_barrier()
    wo = [
        pltpu.async_copy(coeff_sp.at[pl.ds(g * _NP + nbase, _NPT)],
                         out_hbm.at[c, g, pl.ds(nbase, _NPT)], sem_in)
        for g in range(_G)
    ]
    for cp in wo:
        cp.wait()


_gcn_kernel = pl.kernel(
    _gcn_body,
    out_type=jax.ShapeDtypeStruct((_NC, _G, _NP), jnp.float32),
    mesh=_mesh,
    compiler_params=_sc_params,
    scratch_types=[
        pltpu.VMEM((_EAT,), jnp.int32),
        pltpu.VMEM((_EAT,), jnp.float32),
        pltpu.VMEM((_EB + 128,), jnp.int32),
        pltpu.VMEM((_NS, _NPT), jnp.float32),
        pltpu.VMEM((_NPT,), jnp.float32),
        pltpu.VMEM((_NP,), jnp.float32),
        pltpu.VMEM((_NPT,), jnp.int32),
        pltpu.VMEM((_ROWS, 128), jnp.int32),
        pltpu.VMEM((_ROWS, 128), jnp.float32),
        pltpu.VMEM((_NP,), jnp.float32),
        pltpu.VMEM_SHARED((_NP,), jnp.float32),
        pltpu.VMEM_SHARED((_COEFF,), jnp.float32),
        pltpu.SemaphoreType.DMA,
        pltpu.SemaphoreType.DMA,
    ],
)


def _pool_body(c_ref, x_ref, w_ref, bb_ref, batch_ref, out_ref):
    bb = jnp.reshape(bb_ref[...], (1, _C))
    bv = jnp.broadcast_to(jnp.reshape(batch_ref[...], (1, _N)), (_G, _N))
    gi = lax.broadcasted_iota(jnp.int32, (_G, _N), 0)
    cnt = jnp.sum((bv == gi).astype(jnp.float32), axis=1, keepdims=True)
    pos = cnt > 0.0
    invc = jnp.where(pos, 1.0 / jnp.maximum(cnt, 1.0), 0.0)
    bmask = jnp.where(pos, 1.0, 0.0)
    c2 = (c_ref[0] + c_ref[1])[:, :_N]
    s = lax.dot_general(c2, x_ref[...], (((1,), (0,)), ((), ())),
                        preferred_element_type=jnp.float32)
    p = lax.dot_general(s, w_ref[...], (((1,), (0,)), ((), ())),
                        preferred_element_type=jnp.float32)
    out_ref[...] = p * invc + bb * bmask


_pool_kernel = pl.pallas_call(
    _pool_body,
    out_shape=jax.ShapeDtypeStruct((_G, _C), jnp.float32),
)


def kernel(x, edge_index, edge_weights, batch, W, b):
    ei = edge_index.astype(jnp.int32)
    batch32 = batch.astype(jnp.int32)
    w = edge_weights.astype(jnp.float32)

    coeffp = _gcn_kernel(ei, w, batch32)
    return _pool_kernel(coeffp, x, W, b, batch32)
